# baseline (speedup 1.0000x reference)
import functools

import jax
import jax.numpy as jnp
from jax import lax
from jax.experimental import pallas as pl
from jax.experimental.pallas import tpu as pltpu
from jax.experimental.pallas import tpu_sc as plsc

POOLED_H = 7
POOLED_W = 7
SAMPLING = 2
K = SAMPLING * SAMPLING * 4
NUM_CORES = 2
NUM_SUBCORES = 16
LANES = 16
NW = NUM_CORES * NUM_SUBCORES
TBINS = 8


def _prep_body(H, W, scale_ref, rois_ref, idx_ref, w_ref):
    nb = POOLED_H * POOLED_W
    scale = scale_ref[0, 0]
    rois = rois_ref[...]
    b = rois[:, 0:1].astype(jnp.int32)
    x1 = rois[:, 1:2] * scale - 0.5
    y1 = rois[:, 2:3] * scale - 0.5
    x2 = rois[:, 3:4] * scale - 0.5
    y2 = rois[:, 4:5] * scale - 0.5
    bin_w = (x2 - x1) / float(POOLED_W)
    bin_h = (y2 - y1) / float(POOLED_H)
    n = rois.shape[0]
    bi = lax.broadcasted_iota(jnp.int32, (n, nb), 1)
    phf = (bi // POOLED_W).astype(jnp.float32)
    pwf = (bi % POOLED_W).astype(jnp.float32)
    for k in range(K):
        s, corner = k // 4, k % 4
        iy, ix = s // SAMPLING, s % SAMPLING
        cy, cx = corner // 2, corner % 2
        yy = y1 + (phf + (iy + 0.5) / SAMPLING) * bin_h
        xx = x1 + (pwf + (ix + 0.5) / SAMPLING) * bin_w
        valid = ((yy > -1.0) & (yy < float(H)) & (xx > -1.0) & (xx < float(W)))
        yc = jnp.clip(yy, 0.0, float(H - 1))
        xc = jnp.clip(xx, 0.0, float(W - 1))
        y0f = jnp.floor(yc)
        x0f = jnp.floor(xc)
        y0 = y0f.astype(jnp.int32)
        x0 = x0f.astype(jnp.int32)
        ly = yc - y0f
        lx = xc - x0f
        if cy == 0:
            wy, ysel = 1.0 - ly, y0
        else:
            wy, ysel = ly, jnp.minimum(y0 + 1, H - 1)
        if cx == 0:
            wx, xsel = 1.0 - lx, x0
        else:
            wx, xsel = lx, jnp.minimum(x0 + 1, W - 1)
        w = wy * wx * valid.astype(jnp.float32) * (1.0 / (SAMPLING * SAMPLING))
        idx = b * (H * W) + ysel * W + xsel
        idx_ref[k, :, :] = idx
        w_ref[k, :, :] = w


def _sc_body(steps, C, feat_hbm, idx_hbm, w_hbm, out_hbm,
             idx_v, w_v, rows0, rows1, out0, out1,
             gsem0, gsem1, osem0, osem1):
    wid = lax.axis_index("s") * NUM_CORES + lax.axis_index("c")
    base_bin = wid * (TBINS * steps)
    span = steps * TBINS * K

    pltpu.sync_copy(idx_hbm.at[pl.ds(base_bin * K, span)], idx_v)
    pltpu.sync_copy(w_hbm.at[pl.ds(base_bin * K, span)], w_v)

    def gather(s, rows, sem):
        return pltpu.make_async_copy(
            feat_hbm.at[idx_v.at[pl.ds(s * TBINS * K, TBINS * K)]], rows, sem)

    def outcopy(s, out_v, sem):
        return pltpu.make_async_copy(
            out_v, out_hbm.at[pl.ds(base_bin + s * TBINS, TBINS)], sem)

    def compute(s, rows_v, out_v):
        @plsc.parallel_loop(0, TBINS, 1, unroll=4)
        def _bin(t):
            woff = s * (TBINS * K) + t * K
            wv = [
                plsc.load_gather(
                    w_v, [jnp.full((LANES,), woff + k, dtype=jnp.int32)])
                for k in range(K)
            ]
            r0 = t * K
            for c in range(C // LANES):
                sl = pl.ds(c * LANES, LANES)
                acc0 = wv[0] * rows_v[r0, sl]
                acc1 = wv[1] * rows_v[r0 + 1, sl]
                for k in range(2, K, 2):
                    acc0 = acc0 + wv[k] * rows_v[r0 + k, sl]
                    acc1 = acc1 + wv[k + 1] * rows_v[r0 + k + 1, sl]
                out_v[t, sl] = acc0 + acc1

    gather(0, rows0, gsem0).start()
    gather(1, rows1, gsem1).start()

    @pl.loop(0, steps // 2)
    def _pair(i):
        s0 = 2 * i
        for par, rows, out_v, gsem, osem in (
                (0, rows0, out0, gsem0, osem0),
                (1, rows1, out1, gsem1, osem1)):
            s = s0 + par
            gather(s, rows, gsem).wait()

            @pl.when(i > 0)
            def _wait_prev_out():
                outcopy(s - 2, out_v, osem).wait()

            compute(s, rows, out_v)
            outcopy(s, out_v, osem).start()

            @pl.when(s + 2 < steps)
            def _next_gather():
                gather(s + 2, rows, gsem).start()

    outcopy(steps - 2, out0, osem0).wait()
    outcopy(steps - 1, out1, osem1).wait()


def kernel(rois, feature, stride):
    N = rois.shape[0]
    B, C, H, W = feature.shape
    nb = POOLED_H * POOLED_W
    bins = N * nb
    steps = -(-bins // (NW * TBINS))
    steps = steps + (steps % 2)
    bp = NW * TBINS * steps

    scale = (1.0 / jnp.asarray(stride, dtype=jnp.float32)).reshape(1, 1)
    idx3, w3 = pl.pallas_call(
        functools.partial(_prep_body, H, W),
        out_shape=(
            jax.ShapeDtypeStruct((K, N, nb), jnp.int32),
            jax.ShapeDtypeStruct((K, N, nb), jnp.float32),
        ),
        in_specs=[
            pl.BlockSpec(memory_space=pltpu.MemorySpace.SMEM),
            pl.BlockSpec(memory_space=pltpu.MemorySpace.VMEM),
        ],
    )(scale, rois[:, :5].astype(jnp.float32))

    idx_flat = jnp.transpose(idx3, (1, 2, 0)).reshape(bins * K)
    w_flat = jnp.transpose(w3, (1, 2, 0)).reshape(bins * K)
    idx_flat = jnp.pad(idx_flat, (0, (bp - bins) * K))
    w_flat = jnp.pad(w_flat, (0, (bp - bins) * K))

    featT = jnp.transpose(feature, (0, 2, 3, 1)).reshape(B * H * W, C)

    sc_fn = pl.kernel(
        functools.partial(_sc_body, steps, C),
        out_type=jax.ShapeDtypeStruct((bp, C), jnp.float32),
        mesh=plsc.VectorSubcoreMesh(
            core_axis_name="c", subcore_axis_name="s",
            num_cores=NUM_CORES, num_subcores=NUM_SUBCORES),
        scratch_types=[
            pltpu.VMEM((steps * TBINS * K,), jnp.int32),
            pltpu.VMEM((steps * TBINS * K,), jnp.float32),
            pltpu.VMEM((TBINS * K, C), jnp.float32),
            pltpu.VMEM((TBINS * K, C), jnp.float32),
            pltpu.VMEM((TBINS, C), jnp.float32),
            pltpu.VMEM((TBINS, C), jnp.float32),
            pltpu.SemaphoreType.DMA,
            pltpu.SemaphoreType.DMA,
            pltpu.SemaphoreType.DMA,
            pltpu.SemaphoreType.DMA,
        ],
        compiler_params=pltpu.CompilerParams(needs_layout_passes=False),
    )
    out_flat = sc_fn(featT, idx_flat, w_flat)

    out = out_flat[:bins].reshape(N, nb, C)
    out = jnp.transpose(out, (0, 2, 1)).reshape(N, C, POOLED_H, POOLED_W)
    return out


# speedup vs baseline: 1.3706x; 1.3706x over previous
import functools

import jax
import jax.numpy as jnp
from jax import lax
from jax.experimental import pallas as pl
from jax.experimental.pallas import tpu as pltpu
from jax.experimental.pallas import tpu_sc as plsc

POOLED_H = 7
POOLED_W = 7
SAMPLING = 2
K = SAMPLING * SAMPLING * 4
NUM_CORES = 2
NUM_SUBCORES = 16
LANES = 16
NW = NUM_CORES * NUM_SUBCORES
TBINS = 8


def _prep_body(H, W, scale_ref, rois_ref, idx_ref, w_ref):
    nb = POOLED_H * POOLED_W
    scale = scale_ref[0, 0]
    rois = rois_ref[...]
    b = rois[:, 0:1].astype(jnp.int32)
    x1 = rois[:, 1:2] * scale - 0.5
    y1 = rois[:, 2:3] * scale - 0.5
    x2 = rois[:, 3:4] * scale - 0.5
    y2 = rois[:, 4:5] * scale - 0.5
    bin_w = (x2 - x1) / float(POOLED_W)
    bin_h = (y2 - y1) / float(POOLED_H)
    n = rois.shape[0]
    bi = lax.broadcasted_iota(jnp.int32, (n, nb), 1)
    phf = (bi // POOLED_W).astype(jnp.float32)
    pwf = (bi % POOLED_W).astype(jnp.float32)
    for k in range(K):
        s, corner = k // 4, k % 4
        iy, ix = s // SAMPLING, s % SAMPLING
        cy, cx = corner // 2, corner % 2
        yy = y1 + (phf + (iy + 0.5) / SAMPLING) * bin_h
        xx = x1 + (pwf + (ix + 0.5) / SAMPLING) * bin_w
        valid = ((yy > -1.0) & (yy < float(H)) & (xx > -1.0) & (xx < float(W)))
        yc = jnp.clip(yy, 0.0, float(H - 1))
        xc = jnp.clip(xx, 0.0, float(W - 1))
        y0f = jnp.floor(yc)
        x0f = jnp.floor(xc)
        y0 = y0f.astype(jnp.int32)
        x0 = x0f.astype(jnp.int32)
        ly = yc - y0f
        lx = xc - x0f
        if cy == 0:
            wy, ysel = 1.0 - ly, y0
        else:
            wy, ysel = ly, jnp.minimum(y0 + 1, H - 1)
        if cx == 0:
            wx, xsel = 1.0 - lx, x0
        else:
            wx, xsel = lx, jnp.minimum(x0 + 1, W - 1)
        w = wy * wx * valid.astype(jnp.float32) * (1.0 / (SAMPLING * SAMPLING))
        idx = b * (H * W) + ysel * W + xsel
        idx_ref[k, :, :] = idx
        w_ref[k, :, :] = w


def _sc_body(steps, C, feat_hbm, idx_hbm, w_hbm, out_hbm,
             idx_v, w_v, rows0, rows1, out0, out1,
             gsem0, gsem1, osem0, osem1):
    wid = lax.axis_index("s") * NUM_CORES + lax.axis_index("c")
    base_bin = wid * (TBINS * steps)
    span = steps * TBINS * K

    pltpu.sync_copy(idx_hbm.at[pl.ds(base_bin * K, span)], idx_v)
    pltpu.sync_copy(w_hbm.at[pl.ds(base_bin * K, span)], w_v)

    def gather(s, rows, sem):
        return pltpu.make_async_copy(
            feat_hbm.at[idx_v.at[pl.ds(s * TBINS * K, TBINS * K)]], rows, sem)

    def outcopy(s, out_v, sem):
        return pltpu.make_async_copy(
            out_v, out_hbm.at[pl.ds(base_bin + s * TBINS, TBINS)], sem)

    def compute(s, rows_v, out_v):
        @plsc.parallel_loop(0, TBINS, 1, unroll=2)
        def _bin(t):
            woff = s * (TBINS * K) + t * K
            wv = [
                plsc.load_gather(
                    w_v, [jnp.full((LANES,), woff + k, dtype=jnp.int32)])
                for k in range(K)
            ]
            r0 = t * K
            for c in range(C // LANES):
                sl = pl.ds(c * LANES, LANES)
                acc0 = wv[0] * rows_v[r0, sl]
                acc1 = wv[1] * rows_v[r0 + 1, sl]
                for k in range(2, K, 2):
                    acc0 = acc0 + wv[k] * rows_v[r0 + k, sl]
                    acc1 = acc1 + wv[k + 1] * rows_v[r0 + k + 1, sl]
                out_v[t, sl] = acc0 + acc1

    gather(0, rows0, gsem0).start()
    gather(1, rows1, gsem1).start()

    @pl.loop(0, steps // 2)
    def _pair(i):
        s0 = 2 * i
        for par, rows, out_v, gsem, osem in (
                (0, rows0, out0, gsem0, osem0),
                (1, rows1, out1, gsem1, osem1)):
            s = s0 + par
            gather(s, rows, gsem).wait()

            @pl.when(i > 0)
            def _wait_prev_out():
                outcopy(s - 2, out_v, osem).wait()

            compute(s, rows, out_v)
            outcopy(s, out_v, osem).start()

            @pl.when(s + 2 < steps)
            def _next_gather():
                gather(s + 2, rows, gsem).start()

    outcopy(steps - 2, out0, osem0).wait()
    outcopy(steps - 1, out1, osem1).wait()


def kernel(rois, feature, stride):
    N = rois.shape[0]
    B, C, H, W = feature.shape
    nb = POOLED_H * POOLED_W
    bins = N * nb
    steps = -(-bins // (NW * TBINS))
    steps = steps + (steps % 2)
    bp = NW * TBINS * steps

    scale = (1.0 / jnp.asarray(stride, dtype=jnp.float32)).reshape(1, 1)
    idx3, w3 = pl.pallas_call(
        functools.partial(_prep_body, H, W),
        out_shape=(
            jax.ShapeDtypeStruct((K, N, nb), jnp.int32),
            jax.ShapeDtypeStruct((K, N, nb), jnp.float32),
        ),
        in_specs=[
            pl.BlockSpec(memory_space=pltpu.MemorySpace.SMEM),
            pl.BlockSpec(memory_space=pltpu.MemorySpace.VMEM),
        ],
    )(scale, rois[:, :5].astype(jnp.float32))

    idx_flat = jnp.transpose(idx3, (1, 2, 0)).reshape(bins * K)
    w_flat = jnp.transpose(w3, (1, 2, 0)).reshape(bins * K)
    idx_flat = jnp.pad(idx_flat, (0, (bp - bins) * K))
    w_flat = jnp.pad(w_flat, (0, (bp - bins) * K))

    featT = jnp.transpose(feature, (0, 2, 3, 1)).reshape(B * H * W, C)

    sc_fn = pl.kernel(
        functools.partial(_sc_body, steps, C),
        out_type=jax.ShapeDtypeStruct((bp, C), jnp.float32),
        mesh=plsc.VectorSubcoreMesh(
            core_axis_name="c", subcore_axis_name="s",
            num_cores=NUM_CORES, num_subcores=NUM_SUBCORES),
        scratch_types=[
            pltpu.VMEM((steps * TBINS * K,), jnp.int32),
            pltpu.VMEM((steps * TBINS * K,), jnp.float32),
            pltpu.VMEM((TBINS * K, C), jnp.float32),
            pltpu.VMEM((TBINS * K, C), jnp.float32),
            pltpu.VMEM((TBINS, C), jnp.float32),
            pltpu.VMEM((TBINS, C), jnp.float32),
            pltpu.SemaphoreType.DMA,
            pltpu.SemaphoreType.DMA,
            pltpu.SemaphoreType.DMA,
            pltpu.SemaphoreType.DMA,
        ],
        compiler_params=pltpu.CompilerParams(needs_layout_passes=False),
    )
    out_flat = sc_fn(featT, idx_flat, w_flat)

    out = out_flat[:bins].reshape(N, nb, C)
    out = jnp.transpose(out, (0, 2, 1)).reshape(N, C, POOLED_H, POOLED_W)
    return out
